# baseline (speedup 1.0000x reference)
import functools

import jax
import jax.numpy as jnp
from jax import lax
from jax.experimental import pallas as pl
from jax.experimental.pallas import tpu as pltpu
from jax.experimental.pallas import tpu_sc as plsc

ALPHA = 32.0


def _sc_gather(table, flat_len, inner):
    n, d = flat_len.shape[0], table.shape[1]
    info = plsc.get_sparse_core_info()
    num_cores = 1
    nw = num_cores * info.num_subcores
    lanes = info.num_lanes
    n_per_w = n // nw
    mesh = plsc.VectorSubcoreMesh(
        core_axis_name="c", subcore_axis_name="s", num_cores=num_cores
    )

    @functools.partial(
        pl.kernel,
        mesh=mesh,
        out_type=jax.ShapeDtypeStruct((n, d), jnp.float32),
        scratch_types=[
            pltpu.VMEM((n_per_w,), jnp.int32),
            pltpu.VMEM((n_per_w, d), jnp.float32),
        ]
        + [pltpu.SemaphoreType.DMA] * (n_per_w // 16)
        + [pltpu.SemaphoreType.DMA],
    )
    def gather_kernel(table_hbm, len_hbm, out_hbm, len_v, rows_v, *sems):
        in_sems, out_sem = sems[:-1], sems[-1]
        wid = lax.axis_index("s") * num_cores + lax.axis_index("c")
        base = wid * n_per_w
        pltpu.sync_copy(len_hbm.at[pl.ds(base, n_per_w)], len_v)
        copies = []
        for j in range(n_per_w // lanes):
            row0 = base + j * lanes
            idx = (row0 + lax.iota(jnp.int32, 16)) * inner + len_v[
                pl.ds(j * lanes, lanes)
            ]
            copies.append(
                pltpu.async_copy(
                    table_hbm.at[idx],
                    rows_v.at[pl.ds(j * lanes, lanes)],
                    in_sems[j],
                )
            )
        outs = []
        for j, c in enumerate(copies):
            c.wait()
            outs.append(
                pltpu.async_copy(
                    rows_v.at[pl.ds(j * lanes, lanes)],
                    out_hbm.at[pl.ds(base + j * lanes, lanes)],
                    out_sem,
                )
            )
        for c in outs:
            c.wait()

    return gather_kernel(table, flat_len)


def _tc_lora_linear(x, w, b2d, lora_a, lora_b, scaling):
    n, d = x.shape

    def body(x_ref, w_ref, b_ref, a_ref, bb_ref, o_ref):
        m = w_ref[:] + scaling * jnp.dot(
            bb_ref[:], a_ref[:], preferred_element_type=jnp.float32
        )
        o_ref[:] = (
            lax.dot_general(
                x_ref[:], m, (((1,), (1,)), ((), ())),
                preferred_element_type=jnp.float32,
            )
            + b_ref[:]
        )

    return pl.pallas_call(
        body,
        out_shape=jax.ShapeDtypeStruct((n, d), jnp.float32),
    )(x, w, b2d, lora_a, lora_b)


def _tc_fused(ffn_flat, len_flat, w, b2d, lora_a, lora_b, scaling, inner):
    n, d = len_flat.shape[0], ffn_flat.shape[1]

    chunks = 4
    nb = n // chunks

    def body(len_ref, w_ref, b_ref, a_ref, bb_ref, ffn_ref, o_ref, x_ref,
             m_ref, sem0, sem1):
        c = pl.program_id(0)
        sems = [sem0, sem1]

        def issue_chunk(chunk, buf, sem):
            base = chunk * nb
            xb = buf * nb

            def issue(r, _):
                t = (base + r) * inner + len_ref[base + r]
                pltpu.make_async_copy(
                    ffn_ref.at[t], x_ref.at[xb + r], sem
                ).start()
                return 0

            lax.fori_loop(0, nb, issue, 0, unroll=16)

        @pl.when(c == 0)
        def _():
            m_ref[:] = w_ref[:] + scaling * jnp.dot(
                bb_ref[:], a_ref[:], preferred_element_type=jnp.float32
            )
            issue_chunk(0, 0, sem0)

        @pl.when(c + 1 < chunks)
        def _():
            nxt = c + 1
            buf = lax.rem(nxt, 2)
            lax.cond(
                buf == 0,
                lambda: issue_chunk(nxt, 0, sem0),
                lambda: issue_chunk(nxt, 1, sem1),
            )

        cbuf = pl.multiple_of(lax.rem(c, 2) * nb, nb)
        for p, sem in enumerate(sems):
            @pl.when(lax.rem(c, 2) == p)
            def _():
                pltpu.make_async_copy(
                    ffn_ref.at[pl.ds(0, nb)],
                    x_ref.at[pl.ds(p * nb, nb)],
                    sem,
                ).wait()
        o_ref[:] = (
            lax.dot_general(
                x_ref[pl.ds(cbuf, nb), :], m_ref[:], (((1,), (1,)), ((), ())),
                preferred_element_type=jnp.float32,
            )
            + b_ref[:]
        )

    return pl.pallas_call(
        body,
        grid=(chunks,),
        in_specs=[
            pl.BlockSpec(memory_space=pltpu.MemorySpace.SMEM),
            pl.BlockSpec(memory_space=pltpu.MemorySpace.VMEM),
            pl.BlockSpec(memory_space=pltpu.MemorySpace.VMEM),
            pl.BlockSpec(memory_space=pltpu.MemorySpace.VMEM),
            pl.BlockSpec(memory_space=pltpu.MemorySpace.VMEM),
            pl.BlockSpec(memory_space=pltpu.MemorySpace.HBM),
        ],
        out_specs=pl.BlockSpec((nb, d), lambda c: (c, 0)),
        out_shape=jax.ShapeDtypeStruct((n, d), jnp.float32),
        scratch_shapes=[
            pltpu.VMEM((2 * nb, d), jnp.float32),
            pltpu.VMEM((d, d), jnp.float32),
            pltpu.SemaphoreType.DMA,
            pltpu.SemaphoreType.DMA,
        ],
    )(len_flat, w, b2d, lora_a, lora_b, ffn_flat)


def kernel(ffn_out, lengths, W, b, lora_A, lora_B):
    bsz, s, i, d = ffn_out.shape
    r = lora_A.shape[0]
    scaling = ALPHA / r

    table = ffn_out.reshape(bsz * s * i, d)
    out = _tc_fused(
        table,
        lengths.reshape(-1).astype(jnp.int32),
        W,
        b.reshape(1, d),
        lora_A,
        lora_B,
        scaling,
        i,
    )
    return out.reshape(bsz, s, d)


# speedup vs baseline: 1.0574x; 1.0574x over previous
import jax
import jax.numpy as jnp
from jax import lax
from jax.experimental import pallas as pl
from jax.experimental.pallas import tpu as pltpu

ALPHA = 32.0


def _fused_gather_lora_linear(ffn_flat, len_flat, w, b2d, lora_a, lora_b,
                              scaling, inner):
    n, d = len_flat.shape[0], ffn_flat.shape[1]

    def body(len_ref, w_ref, b_ref, a_ref, bb_ref, ffn_ref, o_ref, x_ref, sem):
        def issue(r, _):
            t = r * inner + len_ref[r]
            pltpu.make_async_copy(ffn_ref.at[t], x_ref.at[r], sem).start()
            return 0

        lax.fori_loop(0, n, issue, 0, unroll=16)
        m = w_ref[:] + scaling * jnp.dot(
            bb_ref[:], a_ref[:], preferred_element_type=jnp.float32
        )
        pltpu.make_async_copy(ffn_ref.at[pl.ds(0, n)], x_ref, sem).wait()
        o_ref[:] = (
            lax.dot_general(
                x_ref[:], m, (((1,), (1,)), ((), ())),
                preferred_element_type=jnp.float32,
            )
            + b_ref[:]
        )

    return pl.pallas_call(
        body,
        in_specs=[
            pl.BlockSpec(memory_space=pltpu.MemorySpace.SMEM),
            pl.BlockSpec(memory_space=pltpu.MemorySpace.VMEM),
            pl.BlockSpec(memory_space=pltpu.MemorySpace.VMEM),
            pl.BlockSpec(memory_space=pltpu.MemorySpace.VMEM),
            pl.BlockSpec(memory_space=pltpu.MemorySpace.VMEM),
            pl.BlockSpec(memory_space=pltpu.MemorySpace.HBM),
        ],
        out_specs=pl.BlockSpec(memory_space=pltpu.MemorySpace.VMEM),
        out_shape=jax.ShapeDtypeStruct((n, d), jnp.float32),
        scratch_shapes=[
            pltpu.VMEM((n, d), jnp.float32),
            pltpu.SemaphoreType.DMA,
        ],
    )(len_flat, w, b2d, lora_a, lora_b, ffn_flat)


def kernel(ffn_out, lengths, W, b, lora_A, lora_B):
    bsz, s, i, d = ffn_out.shape
    r = lora_A.shape[0]
    scaling = ALPHA / r

    table = ffn_out.reshape(bsz * s * i, d)
    out = _fused_gather_lora_linear(
        table,
        lengths.reshape(-1).astype(jnp.int32),
        W,
        b.reshape(1, d),
        lora_A,
        lora_B,
        scaling,
        i,
    )
    return out.reshape(bsz, s, d)
